# Initial kernel scaffold; baseline (speedup 1.0000x reference)
import jax
import jax.numpy as jnp
from jax.experimental import pallas as pl


def kernel(weight_probs, weight_indices, bias_probs, bias_indices, weight_bank, bias_bank):
    raise NotImplementedError("write your pallas kernel here")


# speedup vs baseline: 2.3880x; 2.3880x over previous
import jax
import jax.numpy as jnp
from jax.experimental import pallas as pl

N, K, E, O, I = 2048, 2, 64, 128, 128
M = O * I

TN = 512
TM = 2048


def _mix_kernel(wp_ref, wi_ref, bp_ref, bi_ref, bank_ref, bbank_ref,
                out_ref, bout_ref):
    j = pl.program_id(1)
    wp = wp_ref[...]
    wi = wi_ref[...]
    iota = jax.lax.broadcasted_iota(jnp.int32, (TN, E), 1)
    s = (wp[:, 0:1] * (wi[:, 0:1] == iota).astype(jnp.float32)
         + wp[:, 1:2] * (wi[:, 1:2] == iota).astype(jnp.float32))
    out_ref[...] = jnp.dot(s, bank_ref[...], preferred_element_type=jnp.float32)

    @pl.when(j == 0)
    def _():
        bp = bp_ref[...]
        bi = bi_ref[...]
        sb = (bp[:, 0:1] * (bi[:, 0:1] == iota).astype(jnp.float32)
              + bp[:, 1:2] * (bi[:, 1:2] == iota).astype(jnp.float32))
        bout_ref[...] = jnp.dot(sb, bbank_ref[...],
                                preferred_element_type=jnp.float32)


def kernel(weight_probs, weight_indices, bias_probs, bias_indices,
           weight_bank, bias_bank):
    wi = weight_indices.astype(jnp.int32)
    bi = bias_indices.astype(jnp.int32)
    bank2d = weight_bank.reshape(E, M)

    grid = (N // TN, M // TM)
    out, bout = pl.pallas_call(
        _mix_kernel,
        grid=grid,
        in_specs=[
            pl.BlockSpec((TN, K), lambda i, j: (i, 0)),
            pl.BlockSpec((TN, K), lambda i, j: (i, 0)),
            pl.BlockSpec((TN, K), lambda i, j: (i, 0)),
            pl.BlockSpec((TN, K), lambda i, j: (i, 0)),
            pl.BlockSpec((E, TM), lambda i, j: (0, j)),
            pl.BlockSpec((E, O), lambda i, j: (0, 0)),
        ],
        out_specs=[
            pl.BlockSpec((TN, TM), lambda i, j: (i, j)),
            pl.BlockSpec((TN, O), lambda i, j: (i, 0)),
        ],
        out_shape=[
            jax.ShapeDtypeStruct((N, M), jnp.float32),
            jax.ShapeDtypeStruct((N, O), jnp.float32),
        ],
    )(weight_probs, wi, bias_probs, bi, bank2d, bias_bank)

    return out.reshape(N, O, I), bout


# speedup vs baseline: 2.4525x; 1.0270x over previous
import jax
import jax.numpy as jnp
from jax.experimental import pallas as pl

N, K, E, O, I = 2048, 2, 64, 128, 128
M = O * I

TN = 512
TM = 2048


def _mix_kernel(wp_ref, wi_ref, bp_ref, bi_ref, bank_ref, bbank_ref,
                out_ref, bout_ref):
    j = pl.program_id(1)
    wp = wp_ref[...]
    wi = wi_ref[...]
    iota = jax.lax.broadcasted_iota(jnp.int32, (TN, E), 1)
    s = (wp[:, 0:1] * (wi[:, 0:1] == iota).astype(jnp.float32)
         + wp[:, 1:2] * (wi[:, 1:2] == iota).astype(jnp.float32))
    out_ref[...] = jnp.dot(s.astype(jnp.bfloat16), bank_ref[...],
                           preferred_element_type=jnp.float32)

    @pl.when(j == 0)
    def _():
        bp = bp_ref[...]
        bi = bi_ref[...]
        sb = (bp[:, 0:1] * (bi[:, 0:1] == iota).astype(jnp.float32)
              + bp[:, 1:2] * (bi[:, 1:2] == iota).astype(jnp.float32))
        bout_ref[...] = jnp.dot(sb, bbank_ref[...],
                                preferred_element_type=jnp.float32)


def kernel(weight_probs, weight_indices, bias_probs, bias_indices,
           weight_bank, bias_bank):
    wi = weight_indices.astype(jnp.int32)
    bi = bias_indices.astype(jnp.int32)
    bank2d = weight_bank.reshape(E, M).astype(jnp.bfloat16)

    grid = (N // TN, M // TM)
    out, bout = pl.pallas_call(
        _mix_kernel,
        grid=grid,
        in_specs=[
            pl.BlockSpec((TN, K), lambda i, j: (i, 0)),
            pl.BlockSpec((TN, K), lambda i, j: (i, 0)),
            pl.BlockSpec((TN, K), lambda i, j: (i, 0)),
            pl.BlockSpec((TN, K), lambda i, j: (i, 0)),
            pl.BlockSpec((E, TM), lambda i, j: (0, j)),
            pl.BlockSpec((E, O), lambda i, j: (0, 0)),
        ],
        out_specs=[
            pl.BlockSpec((TN, TM), lambda i, j: (i, j)),
            pl.BlockSpec((TN, O), lambda i, j: (i, 0)),
        ],
        out_shape=[
            jax.ShapeDtypeStruct((N, M), jnp.float32),
            jax.ShapeDtypeStruct((N, O), jnp.float32),
        ],
    )(weight_probs, wi, bias_probs, bi, bank2d, bias_bank)

    return out.reshape(N, O, I), bout


# speedup vs baseline: 2.6382x; 1.0758x over previous
import jax
import jax.numpy as jnp
from jax.experimental import pallas as pl

N, K, E, O, I = 2048, 2, 64, 128, 128
M = O * I

TN = 128


def _mix_kernel(wp_ref, wi_ref, bp_ref, bi_ref, bank_ref, bbank_ref,
                out_ref, bout_ref):
    wp = wp_ref[...]
    wi = wi_ref[...]
    iota = jax.lax.broadcasted_iota(jnp.int32, (TN, E), 1)
    s = (wp[:, 0:1] * (wi[:, 0:1] == iota).astype(jnp.float32)
         + wp[:, 1:2] * (wi[:, 1:2] == iota).astype(jnp.float32))
    out_ref[...] = jnp.dot(s.astype(jnp.bfloat16), bank_ref[...],
                           preferred_element_type=jnp.float32)

    bp = bp_ref[...]
    bi = bi_ref[...]
    sb = (bp[:, 0:1] * (bi[:, 0:1] == iota).astype(jnp.float32)
          + bp[:, 1:2] * (bi[:, 1:2] == iota).astype(jnp.float32))
    bout_ref[...] = jnp.dot(sb, bbank_ref[...],
                            preferred_element_type=jnp.float32)


def kernel(weight_probs, weight_indices, bias_probs, bias_indices,
           weight_bank, bias_bank):
    wi = weight_indices.astype(jnp.int32)
    bi = bias_indices.astype(jnp.int32)
    bank2d = weight_bank.reshape(E, M).astype(jnp.bfloat16)

    grid = (N // TN,)
    out, bout = pl.pallas_call(
        _mix_kernel,
        grid=grid,
        in_specs=[
            pl.BlockSpec((TN, K), lambda i: (i, 0)),
            pl.BlockSpec((TN, K), lambda i: (i, 0)),
            pl.BlockSpec((TN, K), lambda i: (i, 0)),
            pl.BlockSpec((TN, K), lambda i: (i, 0)),
            pl.BlockSpec((E, M), lambda i: (0, 0)),
            pl.BlockSpec((E, O), lambda i: (0, 0)),
        ],
        out_specs=[
            pl.BlockSpec((TN, M), lambda i: (i, 0)),
            pl.BlockSpec((TN, O), lambda i: (i, 0)),
        ],
        out_shape=[
            jax.ShapeDtypeStruct((N, M), jnp.float32),
            jax.ShapeDtypeStruct((N, O), jnp.float32),
        ],
    )(weight_probs, wi, bias_probs, bi, bank2d, bias_bank)

    return out.reshape(N, O, I), bout
